# initial kernel scaffold (unmeasured)
import jax
import jax.numpy as jnp
from jax import lax
from jax.experimental import pallas as pl
from jax.experimental.pallas import tpu as pltpu

N_DEV = 4
S = 1024
H_PER = 8
DH = 128
HD = H_PER * DH
SCALE = 0.08838834764831843


def kernel(x, Wq, K_ext, V_ext, Wo):
    idx = lax.axis_index("i")
    x_bf = x[0].astype(jnp.bfloat16)
    wq_bf = Wq.astype(jnp.bfloat16)
    wo_bf = Wo.astype(jnp.bfloat16)
    k_bf = lax.dynamic_index_in_dim(K_ext, idx, 0, keepdims=False).astype(
        jnp.bfloat16
    )
    v_bf = lax.dynamic_index_in_dim(V_ext, idx, 0, keepdims=False).astype(
        jnp.bfloat16
    )

    def body(x_ref, wq_ref, wo_ref, k_ref, v_ref, out_ref,
             wqo_send, wqo_buf, k_buf, v_buf,
             send_sems, recv_sems, kv_sems):
        my = lax.axis_index("i")
        left = lax.rem(my + (N_DEV - 1), N_DEV)
        right = lax.rem(my + 1, N_DEV)

        barrier = pltpu.get_barrier_semaphore()
        for nbr in (left, right):
            pl.semaphore_signal(
                barrier, inc=1, device_id=(nbr,),
                device_id_type=pl.DeviceIdType.MESH,
            )
        pl.semaphore_wait(barrier, 2)

        wqo_send[0] = wq_ref[...]
        wqo_send[1] = wo_ref[...]

        qi = lax.broadcasted_iota(jnp.int32, (S, S), 0)
        ki = lax.broadcasted_iota(jnp.int32, (S, S), 1)
        mask = (jnp.abs(qi - ki) <= 128) | (ki < 32) | (qi < 32)
        bias = jnp.where(mask, 0.0, -1e9)

        rdmas = []
        acc = None
        for h in range(N_DEV):
            j = lax.rem(my - h + N_DEV, N_DEV)
            slot = h % 2

            k_cp = pltpu.make_async_copy(
                k_ref.at[:, pl.ds(j * H_PER, H_PER), :],
                k_buf.at[slot], kv_sems.at[slot, 0])
            v_cp = pltpu.make_async_copy(
                v_ref.at[:, pl.ds(j * H_PER, H_PER), :],
                v_buf.at[slot], kv_sems.at[slot, 1])
            k_cp.start()
            v_cp.start()

            if h > 0:
                rdmas[h - 1].wait_recv()
            if h < N_DEV - 1:
                src = wqo_send if h == 0 else wqo_buf.at[h - 1]
                rdma = pltpu.make_async_remote_copy(
                    src_ref=src,
                    dst_ref=wqo_buf.at[h],
                    send_sem=send_sems.at[h],
                    recv_sem=recv_sems.at[h],
                    device_id=(right,),
                    device_id_type=pl.DeviceIdType.MESH,
                )
                rdma.start()
                rdmas.append(rdma)

            k_cp.wait()
            v_cp.wait()

            if h == 0:
                wq_c = wqo_send[0]
                wo_c = wqo_send[1]
            else:
                wq_c = wqo_buf[h - 1, 0]
                wo_c = wqo_buf[h - 1, 1]

            q = lax.dot_general(
                x_ref[...], wq_c, (((1,), (0,)), ((), ())),
                preferred_element_type=jnp.float32,
            ).astype(jnp.bfloat16)

            ctx_parts = []
            for hh in range(H_PER):
                qh = q[:, hh * DH:(hh + 1) * DH]
                kh = k_buf[slot, :, hh, :]
                s = lax.dot_general(
                    qh, kh, (((1,), (1,)), ((), ())),
                    preferred_element_type=jnp.float32,
                )
                s = s * SCALE + bias
                m = jnp.max(s, axis=1, keepdims=True)
                e = jnp.exp(s - m)
                d = jnp.sum(e, axis=1, keepdims=True)
                vh = v_buf[slot, :, hh, :]
                ctx = lax.dot_general(
                    e.astype(jnp.bfloat16), vh, (((1,), (0,)), ((), ())),
                    preferred_element_type=jnp.float32,
                )
                ctx = ctx / d
                ctx_parts.append(ctx.astype(jnp.bfloat16))
            ctx_full = jnp.concatenate(ctx_parts, axis=1)
            part = lax.dot_general(
                ctx_full, wo_c, (((1,), (0,)), ((), ())),
                preferred_element_type=jnp.float32,
            )
            acc = part if acc is None else acc + part

        out_ref[0] = acc
        for r in rdmas:
            r.wait_send()

    return pl.pallas_call(
        body,
        out_shape=jax.ShapeDtypeStruct((1, S, 1024), jnp.float32),
        in_specs=[
            pl.BlockSpec(memory_space=pltpu.VMEM),
            pl.BlockSpec(memory_space=pltpu.VMEM),
            pl.BlockSpec(memory_space=pltpu.VMEM),
            pl.BlockSpec(memory_space=pltpu.ANY),
            pl.BlockSpec(memory_space=pltpu.ANY),
        ],
        out_specs=pl.BlockSpec(memory_space=pltpu.VMEM),
        scratch_shapes=[
            pltpu.VMEM((2, HD, 1024), jnp.bfloat16),
            pltpu.VMEM((N_DEV - 1, 2, HD, 1024), jnp.bfloat16),
            pltpu.VMEM((2, S, H_PER, DH), jnp.bfloat16),
            pltpu.VMEM((2, S, H_PER, DH), jnp.bfloat16),
            pltpu.SemaphoreType.DMA((N_DEV - 1,)),
            pltpu.SemaphoreType.DMA((N_DEV - 1,)),
            pltpu.SemaphoreType.DMA((2, 2)),
        ],
        compiler_params=pltpu.CompilerParams(collective_id=0),
    )(x_bf, wq_bf, wo_bf, k_bf, v_bf)


# baseline (device time: 219147 ns/iter reference)
import jax
import jax.numpy as jnp
from jax import lax
from jax.experimental import pallas as pl
from jax.experimental.pallas import tpu as pltpu

N_DEV = 4
S = 1024
H_PER = 8
DH = 128
HD = H_PER * DH
SCALE = 0.08838834764831843


def kernel(x, Wq, K_ext, V_ext, Wo):
    idx = lax.axis_index("i")
    x_bf = x[0].astype(jnp.bfloat16)
    wq_bf = Wq.astype(jnp.bfloat16)
    wo_bf = Wo.astype(jnp.bfloat16)
    k_bf = (
        lax.dynamic_index_in_dim(K_ext, idx, 0, keepdims=False)
        .astype(jnp.bfloat16)
        .transpose(1, 0, 2)
    )
    v_bf = (
        lax.dynamic_index_in_dim(V_ext, idx, 0, keepdims=False)
        .astype(jnp.bfloat16)
        .transpose(1, 0, 2)
    )

    def body(x_ref, wq_ref, wo_ref, k_ref, v_ref, out_ref,
             wqo_send, wqo_buf, k_buf, v_buf, mask_buf, q_buf, ctx_buf,
             send_sems, recv_sems, kv_sems):
        my = lax.axis_index("i")
        left = lax.rem(my + (N_DEV - 1), N_DEV)
        right = lax.rem(my + 1, N_DEV)

        barrier = pltpu.get_barrier_semaphore()
        for nbr in (left, right):
            pl.semaphore_signal(
                barrier, inc=1, device_id=(nbr,),
                device_id_type=pl.DeviceIdType.MESH,
            )
        pl.semaphore_wait(barrier, 2)

        wqo_send[0] = wq_ref[...]
        wqo_send[1] = wo_ref[...]

        qi = lax.broadcasted_iota(jnp.int32, (S, S), 0)
        ki = lax.broadcasted_iota(jnp.int32, (S, S), 1)
        mask = (jnp.abs(qi - ki) <= 128) | (ki < 32) | (qi < 32)
        mask_buf[...] = mask.astype(jnp.int8)

        rdmas = []
        for h in range(N_DEV):
            j = lax.rem(my - h + N_DEV, N_DEV)
            slot = h % 2

            k_cp = pltpu.make_async_copy(
                k_ref.at[pl.ds(j * H_PER, H_PER)],
                k_buf.at[slot], kv_sems.at[slot, 0])
            v_cp = pltpu.make_async_copy(
                v_ref.at[pl.ds(j * H_PER, H_PER)],
                v_buf.at[slot], kv_sems.at[slot, 1])
            k_cp.start()
            v_cp.start()

            if h > 0:
                rdmas[h - 1].wait_recv()
            if h < N_DEV - 1:
                src = wqo_send if h == 0 else wqo_buf.at[h - 1]
                rdma = pltpu.make_async_remote_copy(
                    src_ref=src,
                    dst_ref=wqo_buf.at[h],
                    send_sem=send_sems.at[h],
                    recv_sem=recv_sems.at[h],
                    device_id=(right,),
                    device_id_type=pl.DeviceIdType.MESH,
                )
                rdma.start()
                rdmas.append(rdma)

            k_cp.wait()
            v_cp.wait()

            if h == 0:
                wq_c = wqo_send[0]
                wo_c = wqo_send[1]
            else:
                wq_c = wqo_buf[h - 1, 0]
                wo_c = wqo_buf[h - 1, 1]

            q = lax.dot_general(
                x_ref[...], wq_c, (((1,), (0,)), ((), ())),
                preferred_element_type=jnp.float32,
            ).astype(jnp.bfloat16)
            for hh in range(H_PER):
                q_buf[hh] = q[:, hh * DH:(hh + 1) * DH]

            def head_body(hh, carry):
                qh = q_buf[hh]
                kh = k_buf[slot, hh]
                s = lax.dot_general(
                    qh, kh, (((1,), (1,)), ((), ())),
                    preferred_element_type=jnp.float32,
                )
                s = jnp.where(mask_buf[...] != 0, s * SCALE, -1e9)
                m = jnp.max(s, axis=1, keepdims=True)
                e = jnp.exp(s - m).astype(jnp.bfloat16)
                d = jnp.sum(e.astype(jnp.float32), axis=1, keepdims=True)
                vh = v_buf[slot, hh]
                ctx = lax.dot_general(
                    e, vh, (((1,), (0,)), ((), ())),
                    preferred_element_type=jnp.float32,
                )
                ctx_buf[hh] = (ctx / d).astype(jnp.bfloat16)
                return carry

            lax.fori_loop(0, H_PER, head_body, 0)

            ctx_full = jnp.concatenate(
                [ctx_buf[i] for i in range(H_PER)], axis=1
            )
            part = lax.dot_general(
                ctx_full, wo_c, (((1,), (0,)), ((), ())),
                preferred_element_type=jnp.float32,
            )
            if h == 0:
                out_ref[0] = part
            else:
                out_ref[0] += part

        for r in rdmas:
            r.wait_send()

    return pl.pallas_call(
        body,
        out_shape=jax.ShapeDtypeStruct((1, S, 1024), jnp.float32),
        in_specs=[
            pl.BlockSpec(memory_space=pltpu.VMEM),
            pl.BlockSpec(memory_space=pltpu.VMEM),
            pl.BlockSpec(memory_space=pltpu.VMEM),
            pl.BlockSpec(memory_space=pl.ANY),
            pl.BlockSpec(memory_space=pl.ANY),
        ],
        out_specs=pl.BlockSpec(memory_space=pltpu.VMEM),
        scratch_shapes=[
            pltpu.VMEM((2, HD, 1024), jnp.bfloat16),
            pltpu.VMEM((N_DEV - 1, 2, HD, 1024), jnp.bfloat16),
            pltpu.VMEM((2, H_PER, S, DH), jnp.bfloat16),
            pltpu.VMEM((2, H_PER, S, DH), jnp.bfloat16),
            pltpu.VMEM((S, S), jnp.int8),
            pltpu.VMEM((H_PER, S, DH), jnp.bfloat16),
            pltpu.VMEM((H_PER, S, DH), jnp.bfloat16),
            pltpu.SemaphoreType.DMA((N_DEV - 1,)),
            pltpu.SemaphoreType.DMA((N_DEV - 1,)),
            pltpu.SemaphoreType.DMA((2, 2)),
        ],
        compiler_params=pltpu.CompilerParams(
            collective_id=0,
            vmem_limit_bytes=100 * 1024 * 1024,
        ),
    )(x_bf, wq_bf, wo_bf, k_bf, v_bf)


# device time: 209655 ns/iter; 1.0453x vs baseline; 1.0453x over previous
import jax
import jax.numpy as jnp
from jax import lax
from jax.experimental import pallas as pl
from jax.experimental.pallas import tpu as pltpu

N_DEV = 4
S = 1024
H_PER = 8
DH = 128
HD = H_PER * DH
SCALE = 0.08838834764831843


def kernel(x, Wq, K_ext, V_ext, Wo):
    idx = lax.axis_index("i")
    x_bf = x[0].astype(jnp.bfloat16)
    wq_bf = Wq.astype(jnp.bfloat16)
    wo_bf = Wo.astype(jnp.bfloat16)
    k_bf = (
        lax.dynamic_index_in_dim(K_ext, idx, 0, keepdims=False)
        .astype(jnp.bfloat16)
        .transpose(1, 0, 2)
    )
    v_bf = (
        lax.dynamic_index_in_dim(V_ext, idx, 0, keepdims=False)
        .astype(jnp.bfloat16)
        .transpose(1, 0, 2)
    )

    def body(x_ref, wq_ref, wo_ref, k_ref, v_ref, out_ref,
             wqo_send, wqo_buf, k_buf, v_buf, mask_buf, q_buf, ctx_buf,
             send_sems, recv_sems, kv_sems):
        my = lax.axis_index("i")
        left = lax.rem(my + (N_DEV - 1), N_DEV)
        right = lax.rem(my + 1, N_DEV)

        barrier = pltpu.get_barrier_semaphore()
        for nbr in (left, right):
            pl.semaphore_signal(
                barrier, inc=1, device_id=(nbr,),
                device_id_type=pl.DeviceIdType.MESH,
            )
        pl.semaphore_wait(barrier, 2)

        wqo_send[0] = wq_ref[...] * jnp.bfloat16(SCALE)
        wqo_send[1] = wo_ref[...]

        qi = lax.broadcasted_iota(jnp.int32, (S, S), 0)
        ki = lax.broadcasted_iota(jnp.int32, (S, S), 1)
        mask = (jnp.abs(qi - ki) <= 128) | (ki < 32) | (qi < 32)
        mask_buf[...] = mask.astype(jnp.int8)

        rdmas = []
        for h in range(N_DEV):
            j = lax.rem(my - h + N_DEV, N_DEV)
            slot = h % 2

            k_cp = pltpu.make_async_copy(
                k_ref.at[pl.ds(j * H_PER, H_PER)],
                k_buf.at[slot], kv_sems.at[slot, 0])
            v_cp = pltpu.make_async_copy(
                v_ref.at[pl.ds(j * H_PER, H_PER)],
                v_buf.at[slot], kv_sems.at[slot, 1])
            k_cp.start()
            v_cp.start()

            if h > 0:
                rdmas[h - 1].wait_recv()
            if h < N_DEV - 1:
                src = wqo_send if h == 0 else wqo_buf.at[h - 1]
                rdma = pltpu.make_async_remote_copy(
                    src_ref=src,
                    dst_ref=wqo_buf.at[h],
                    send_sem=send_sems.at[h],
                    recv_sem=recv_sems.at[h],
                    device_id=(right,),
                    device_id_type=pl.DeviceIdType.MESH,
                )
                rdma.start()
                rdmas.append(rdma)

            k_cp.wait()
            v_cp.wait()

            if h == 0:
                wq_c = wqo_send[0]
                wo_c = wqo_send[1]
            else:
                wq_c = wqo_buf[h - 1, 0]
                wo_c = wqo_buf[h - 1, 1]

            q = lax.dot_general(
                x_ref[...], wq_c, (((1,), (0,)), ((), ())),
                preferred_element_type=jnp.float32,
            ).astype(jnp.bfloat16)
            for hh in range(H_PER):
                q_buf[hh] = q[:, hh * DH:(hh + 1) * DH]

            def head_body(hh, carry):
                qh = q_buf[hh]
                kh = k_buf[slot, hh]
                s = lax.dot_general(
                    qh, kh, (((1,), (1,)), ((), ())),
                    preferred_element_type=jnp.float32,
                )
                s = jnp.where(mask_buf[...] != 0, s, -1e9)
                e = jnp.exp(s).astype(jnp.bfloat16)
                d = jnp.sum(e.astype(jnp.float32), axis=1, keepdims=True)
                vh = v_buf[slot, hh]
                ctx = lax.dot_general(
                    e, vh, (((1,), (0,)), ((), ())),
                    preferred_element_type=jnp.float32,
                )
                ctx_buf[hh] = (ctx / d).astype(jnp.bfloat16)
                return carry

            lax.fori_loop(0, H_PER, head_body, 0)

            ctx_full = jnp.concatenate(
                [ctx_buf[i] for i in range(H_PER)], axis=1
            )
            part = lax.dot_general(
                ctx_full, wo_c, (((1,), (0,)), ((), ())),
                preferred_element_type=jnp.float32,
            )
            if h == 0:
                out_ref[0] = part
            else:
                out_ref[0] += part

        for r in rdmas:
            r.wait_send()

    return pl.pallas_call(
        body,
        out_shape=jax.ShapeDtypeStruct((1, S, 1024), jnp.float32),
        in_specs=[
            pl.BlockSpec(memory_space=pltpu.VMEM),
            pl.BlockSpec(memory_space=pltpu.VMEM),
            pl.BlockSpec(memory_space=pltpu.VMEM),
            pl.BlockSpec(memory_space=pl.ANY),
            pl.BlockSpec(memory_space=pl.ANY),
        ],
        out_specs=pl.BlockSpec(memory_space=pltpu.VMEM),
        scratch_shapes=[
            pltpu.VMEM((2, HD, 1024), jnp.bfloat16),
            pltpu.VMEM((N_DEV - 1, 2, HD, 1024), jnp.bfloat16),
            pltpu.VMEM((2, H_PER, S, DH), jnp.bfloat16),
            pltpu.VMEM((2, H_PER, S, DH), jnp.bfloat16),
            pltpu.VMEM((S, S), jnp.int8),
            pltpu.VMEM((H_PER, S, DH), jnp.bfloat16),
            pltpu.VMEM((H_PER, S, DH), jnp.bfloat16),
            pltpu.SemaphoreType.DMA((N_DEV - 1,)),
            pltpu.SemaphoreType.DMA((N_DEV - 1,)),
            pltpu.SemaphoreType.DMA((2, 2)),
        ],
        compiler_params=pltpu.CompilerParams(
            collective_id=0,
            vmem_limit_bytes=100 * 1024 * 1024,
        ),
    )(x_bf, wq_bf, wo_bf, k_bf, v_bf)
